# baseline (speedup 1.0000x reference)
import functools

import jax
import jax.numpy as jnp
from jax import lax
from jax.experimental import pallas as pl
from jax.experimental.pallas import tpu as pltpu
from jax.experimental.pallas import tpu_sc as plsc

N_STEPS = 6

NC = 2
NS = 16
CHUNK = 128


def _hall_body(h_ref, w_ref, b_ref, out_ref):
    w = w_ref[0]
    out_ref[0] = lax.dot_general(
        h_ref[...], w, (((1,), (1,)), ((), ())),
        preferred_element_type=jnp.float32) + b_ref[0]


def _hall_call(h, W_msg, b_msg, *, n_blk):
    N, D = h.shape
    K = W_msg.shape[0]
    nb = N // n_blk
    return pl.pallas_call(
        _hall_body,
        grid=(nb, K),
        in_specs=[
            pl.BlockSpec((n_blk, D), lambda i, k: (i, 0)),
            pl.BlockSpec((1, D, D), lambda i, k: (k, 0, 0)),
            pl.BlockSpec((1, 1, D), lambda i, k: (k, 0, 0)),
        ],
        out_specs=pl.BlockSpec((1, n_blk, D), lambda i, k: (k, i, 0)),
        out_shape=jax.ShapeDtypeStruct((K, N, D), jnp.float32),
    )(h, W_msg, b_msg[:, None, :])


NBUF = 2


def _sc_body(n_pad, gpt, hall_ref, gidx_ref, dst_ref, zeros_ref,
             out_ref, idx_v, dst_v, buf_v, gsem, ssem, acc_sh):
    c = lax.axis_index("c")
    s = lax.axis_index("s")
    wid = s * NC + c

    rows_per_tile = n_pad // NS
    pltpu.sync_copy(zeros_ref.at[pl.ds(s * rows_per_tile, rows_per_tile)],
                    acc_sh.at[pl.ds(s * rows_per_tile, rows_per_tile)])
    plsc.subcore_barrier()

    def body(g, carry):
        gg = wid * gpt + g
        pltpu.sync_copy(gidx_ref.at[gg], idx_v)
        pltpu.sync_copy(dst_ref.at[gg], dst_v)
        gathers = [
            pltpu.async_copy(hall_ref.at[idx_v.at[b]], buf_v.at[b], gsem)
            for b in range(NBUF)
        ]
        for g_ in gathers:
            g_.wait()
        scatters = [
            pltpu.async_copy(buf_v.at[b], acc_sh.at[dst_v.at[b]], ssem,
                             add=True)
            for b in range(NBUF)
        ]
        for sc in scatters:
            sc.wait()
        return carry

    lax.fori_loop(0, gpt, body, 0)
    plsc.subcore_barrier()

    pltpu.sync_copy(acc_sh.at[pl.ds(s * rows_per_tile, rows_per_tile)],
                    out_ref.at[c, pl.ds(s * rows_per_tile, rows_per_tile)])


def _sc_call(hall_flat, gidx3, dst3, zeros_np, *, n, d, n_pad, gpt):
    mesh = plsc.VectorSubcoreMesh(core_axis_name="c", subcore_axis_name="s")
    body = functools.partial(_sc_body, n_pad, gpt)
    return pl.kernel(
        body,
        out_type=jax.ShapeDtypeStruct((NC, n_pad, d), jnp.float32),
        mesh=mesh,
        scratch_types=[
            pltpu.VMEM((NBUF, CHUNK), jnp.int32),
            pltpu.VMEM((NBUF, CHUNK), jnp.int32),
            pltpu.VMEM((NBUF, CHUNK, d), jnp.float32),
            pltpu.SemaphoreType.DMA,
            pltpu.SemaphoreType.DMA,
            pltpu.VMEM_SHARED((n_pad, d), jnp.float32),
        ],
    )(hall_flat, gidx3, dst3, zeros_np)


def _gru_body(a01_ref, h_ref, wih_ref, whh_ref, bih_ref, bhh_ref,
              wcls_ref, bcls_ref, hnew_ref, logit_ref):
    i = pl.program_id(0)
    nb = pl.num_programs(0)
    a = a01_ref[0] + a01_ref[1]
    h = h_ref[...]
    gi = lax.dot_general(a, wih_ref[...], (((1,), (1,)), ((), ())),
                         preferred_element_type=jnp.float32) + bih_ref[...]
    gh = lax.dot_general(h, whh_ref[...], (((1,), (1,)), ((), ())),
                         preferred_element_type=jnp.float32) + bhh_ref[...]
    D = h.shape[1]
    r = jax.nn.sigmoid(gi[:, :D] + gh[:, :D])
    z = jax.nn.sigmoid(gi[:, D:2 * D] + gh[:, D:2 * D])
    n = jnp.tanh(gi[:, 2 * D:] + r * gh[:, 2 * D:])
    hn = (1.0 - z) * n + z * h
    hnew_ref[...] = hn

    @pl.when(i == 0)
    def _():
        logit_ref[...] = jnp.zeros_like(logit_ref)

    logit_ref[...] += jnp.sum(hn, axis=0, keepdims=True)

    @pl.when(i == nb - 1)
    def _():
        hg = logit_ref[...]
        logit_ref[...] = lax.dot_general(
            hg, wcls_ref[...], (((1,), (1,)), ((), ())),
            preferred_element_type=jnp.float32) + bcls_ref[...]


def _gru_call(a01, h, W_ih, W_hh, b_ih, b_hh, wcls_pad, bcls_pad, *, n_blk):
    N, D = h.shape
    nb = N // n_blk
    return pl.pallas_call(
        _gru_body,
        grid=(nb,),
        in_specs=[
            pl.BlockSpec((2, n_blk, D), lambda i: (0, i, 0)),
            pl.BlockSpec((n_blk, D), lambda i: (i, 0)),
            pl.BlockSpec((3 * D, D), lambda i: (0, 0)),
            pl.BlockSpec((3 * D, D), lambda i: (0, 0)),
            pl.BlockSpec((1, 3 * D), lambda i: (0, 0)),
            pl.BlockSpec((1, 3 * D), lambda i: (0, 0)),
            pl.BlockSpec((D, D), lambda i: (0, 0)),
            pl.BlockSpec((1, D), lambda i: (0, 0)),
        ],
        out_specs=[
            pl.BlockSpec((n_blk, D), lambda i: (i, 0)),
            pl.BlockSpec((1, D), lambda i: (0, 0)),
        ],
        out_shape=[
            jax.ShapeDtypeStruct((N, D), jnp.float32),
            jax.ShapeDtypeStruct((1, D), jnp.float32),
        ],
    )(a01, h, W_ih, W_hh, b_ih, b_hh, wcls_pad, bcls_pad)


def kernel(x, edge_index, edge_type, W_msg, b_msg, W_ih, W_hh, b_ih, b_hh,
           W_cls, b_cls):
    N, D = x.shape
    K = W_msg.shape[0]
    E = edge_index.shape[1]
    n_cls = W_cls.shape[0]

    src = edge_index[0]
    dst = edge_index[1]
    gidx = edge_type * N + src

    n_workers = NC * NS
    grp = n_workers * NBUF * CHUNK
    e_pad = ((E + grp - 1) // grp) * grp
    gpt = e_pad // grp
    gidx = jnp.concatenate([gidx, jnp.zeros((e_pad - E,), jnp.int32)])
    dst = jnp.concatenate([dst, jnp.full((e_pad - E,), N, jnp.int32)])
    gidx3 = gidx.reshape(e_pad // (NBUF * CHUNK), NBUF, CHUNK)
    dst3 = dst.reshape(e_pad // (NBUF * CHUNK), NBUF, CHUNK)

    n_pad = ((N + 1 + NS * 8 - 1) // (NS * 8)) * (NS * 8)
    zeros_np = jnp.zeros((n_pad, D), jnp.float32)

    bih2 = b_ih.reshape(1, 3 * D)
    bhh2 = b_hh.reshape(1, 3 * D)
    wcls_pad = jnp.zeros((D, D), jnp.float32).at[:n_cls].set(W_cls)
    bcls_pad = jnp.zeros((1, D), jnp.float32).at[0, :n_cls].set(b_cls)

    n_blk = 1000
    h = x
    logits = None
    for _ in range(N_STEPS):
        hall = _hall_call(h, W_msg, b_msg, n_blk=n_blk)
        a01 = _sc_call(hall.reshape(K * N, D), gidx3, dst3, zeros_np,
                       n=N, d=D, n_pad=n_pad, gpt=gpt)
        h, logits = _gru_call(a01, h, W_ih, W_hh, bih2, bhh2,
                              wcls_pad, bcls_pad, n_blk=n_blk)
    return logits[:, :n_cls]


# speedup vs baseline: 1.6843x; 1.6843x over previous
import functools

import jax
import jax.numpy as jnp
from jax import lax
from jax.experimental import pallas as pl
from jax.experimental.pallas import tpu as pltpu
from jax.experimental.pallas import tpu_sc as plsc

N_STEPS = 6

NC = 2
NS = 16
CHUNK = 120


def _hall_body(h_ref, w_ref, b_ref, out_ref):
    w = w_ref[0]
    out_ref[0] = lax.dot_general(
        h_ref[...], w, (((1,), (1,)), ((), ())),
        preferred_element_type=jnp.float32) + b_ref[0]


def _hall_call(h, W_msg, b_msg, *, n_blk):
    N, D = h.shape
    K = W_msg.shape[0]
    nb = N // n_blk
    return pl.pallas_call(
        _hall_body,
        grid=(nb, K),
        in_specs=[
            pl.BlockSpec((n_blk, D), lambda i, k: (i, 0)),
            pl.BlockSpec((1, D, D), lambda i, k: (k, 0, 0)),
            pl.BlockSpec((1, 1, D), lambda i, k: (k, 0, 0)),
        ],
        out_specs=pl.BlockSpec((1, n_blk, D), lambda i, k: (k, i, 0)),
        out_shape=jax.ShapeDtypeStruct((K, N, D), jnp.float32),
    )(h, W_msg, b_msg[:, None, :])


RING = 3


def _sc_body(n_pad, gpt, hall_ref, gidx_ref, dst_ref, zeros_ref,
             out_ref, idx_v, dst_v, buf_v,
             gsem0, gsem1, gsem2, ssem0, ssem1, ssem2, acc_sh):
    gsems = (gsem0, gsem1, gsem2)
    ssems = (ssem0, ssem1, ssem2)
    c = lax.axis_index("c")
    s = lax.axis_index("s")
    wid = s * NC + c

    rows_per_tile = n_pad // NS
    pltpu.sync_copy(zeros_ref.at[pl.ds(s * rows_per_tile, rows_per_tile)],
                    acc_sh.at[pl.ds(s * rows_per_tile, rows_per_tile)])
    plsc.subcore_barrier()

    def body(p, carry):
        gg = wid * gpt + p
        pr = lax.rem(p, 2)
        pltpu.sync_copy(gidx_ref.at[gg], idx_v)
        pltpu.sync_copy(dst_ref.at[gg], dst_v.at[pr])
        gathers = []
        for b in range(RING):
            @pl.when(p > 0)
            def _(b=b):
                pltpu.make_async_copy(zeros_ref.at[pl.ds(0, CHUNK)],
                                      buf_v.at[b], ssems[b]).wait()
            gathers.append(
                pltpu.async_copy(hall_ref.at[idx_v.at[b]], buf_v.at[b],
                                 gsems[b]))
        for b in range(RING):
            gathers[b].wait()
            pltpu.async_copy(buf_v.at[b], acc_sh.at[dst_v.at[pr, b]],
                             ssems[b], add=True)
        return carry

    lax.fori_loop(0, gpt, body, 0)
    for b in range(RING):
        pltpu.make_async_copy(zeros_ref.at[pl.ds(0, CHUNK)],
                              buf_v.at[b], ssems[b]).wait()
    plsc.subcore_barrier()

    pltpu.sync_copy(acc_sh.at[pl.ds(s * rows_per_tile, rows_per_tile)],
                    out_ref.at[c, pl.ds(s * rows_per_tile, rows_per_tile)])


def _sc_call(hall_flat, gidx3, dst3, zeros_np, *, n, d, n_pad, gpt):
    mesh = plsc.VectorSubcoreMesh(core_axis_name="c", subcore_axis_name="s")
    body = functools.partial(_sc_body, n_pad, gpt)
    return pl.kernel(
        body,
        out_type=jax.ShapeDtypeStruct((NC, n_pad, d), jnp.float32),
        mesh=mesh,
        scratch_types=[
            pltpu.VMEM((RING, CHUNK), jnp.int32),
            pltpu.VMEM((2, RING, CHUNK), jnp.int32),
            pltpu.VMEM((RING, CHUNK, d), jnp.float32),
            pltpu.SemaphoreType.DMA,
            pltpu.SemaphoreType.DMA,
            pltpu.SemaphoreType.DMA,
            pltpu.SemaphoreType.DMA,
            pltpu.SemaphoreType.DMA,
            pltpu.SemaphoreType.DMA,
            pltpu.VMEM_SHARED((n_pad, d), jnp.float32),
        ],
    )(hall_flat, gidx3, dst3, zeros_np)


def _gru_body(a01_ref, h_ref, wih_ref, whh_ref, bih_ref, bhh_ref,
              wcls_ref, bcls_ref, hnew_ref, logit_ref):
    i = pl.program_id(0)
    nb = pl.num_programs(0)
    a = a01_ref[0] + a01_ref[1]
    h = h_ref[...]
    gi = lax.dot_general(a, wih_ref[...], (((1,), (1,)), ((), ())),
                         preferred_element_type=jnp.float32) + bih_ref[...]
    gh = lax.dot_general(h, whh_ref[...], (((1,), (1,)), ((), ())),
                         preferred_element_type=jnp.float32) + bhh_ref[...]
    D = h.shape[1]
    r = jax.nn.sigmoid(gi[:, :D] + gh[:, :D])
    z = jax.nn.sigmoid(gi[:, D:2 * D] + gh[:, D:2 * D])
    n = jnp.tanh(gi[:, 2 * D:] + r * gh[:, 2 * D:])
    hn = (1.0 - z) * n + z * h
    hnew_ref[...] = hn

    @pl.when(i == 0)
    def _():
        logit_ref[...] = jnp.zeros_like(logit_ref)

    logit_ref[...] += jnp.sum(hn, axis=0, keepdims=True)

    @pl.when(i == nb - 1)
    def _():
        hg = logit_ref[...]
        logit_ref[...] = lax.dot_general(
            hg, wcls_ref[...], (((1,), (1,)), ((), ())),
            preferred_element_type=jnp.float32) + bcls_ref[...]


def _gru_call(a01, h, W_ih, W_hh, b_ih, b_hh, wcls_pad, bcls_pad, *, n_blk):
    N, D = h.shape
    nb = N // n_blk
    return pl.pallas_call(
        _gru_body,
        grid=(nb,),
        in_specs=[
            pl.BlockSpec((2, n_blk, D), lambda i: (0, i, 0)),
            pl.BlockSpec((n_blk, D), lambda i: (i, 0)),
            pl.BlockSpec((3 * D, D), lambda i: (0, 0)),
            pl.BlockSpec((3 * D, D), lambda i: (0, 0)),
            pl.BlockSpec((1, 3 * D), lambda i: (0, 0)),
            pl.BlockSpec((1, 3 * D), lambda i: (0, 0)),
            pl.BlockSpec((D, D), lambda i: (0, 0)),
            pl.BlockSpec((1, D), lambda i: (0, 0)),
        ],
        out_specs=[
            pl.BlockSpec((n_blk, D), lambda i: (i, 0)),
            pl.BlockSpec((1, D), lambda i: (0, 0)),
        ],
        out_shape=[
            jax.ShapeDtypeStruct((N, D), jnp.float32),
            jax.ShapeDtypeStruct((1, D), jnp.float32),
        ],
    )(a01, h, W_ih, W_hh, b_ih, b_hh, wcls_pad, bcls_pad)


def kernel(x, edge_index, edge_type, W_msg, b_msg, W_ih, W_hh, b_ih, b_hh,
           W_cls, b_cls):
    N, D = x.shape
    K = W_msg.shape[0]
    E = edge_index.shape[1]
    n_cls = W_cls.shape[0]

    src = edge_index[0]
    dst = edge_index[1]
    gidx = edge_type * N + src

    n_workers = NC * NS
    grp = n_workers * RING * CHUNK
    e_pad = ((E + grp - 1) // grp) * grp
    gpt = e_pad // grp
    gidx = jnp.concatenate([gidx, jnp.zeros((e_pad - E,), jnp.int32)])
    dst = jnp.concatenate([dst, jnp.full((e_pad - E,), N, jnp.int32)])
    gidx3 = gidx.reshape(e_pad // (RING * CHUNK), RING, CHUNK)
    dst3 = dst.reshape(e_pad // (RING * CHUNK), RING, CHUNK)

    n_pad = ((N + 1 + NS * 8 - 1) // (NS * 8)) * (NS * 8)
    zeros_np = jnp.zeros((n_pad, D), jnp.float32)

    bih2 = b_ih.reshape(1, 3 * D)
    bhh2 = b_hh.reshape(1, 3 * D)
    wcls_pad = jnp.zeros((D, D), jnp.float32).at[:n_cls].set(W_cls)
    bcls_pad = jnp.zeros((1, D), jnp.float32).at[0, :n_cls].set(b_cls)

    n_blk = 1000
    h = x
    logits = None
    for _ in range(N_STEPS):
        hall = _hall_call(h, W_msg, b_msg, n_blk=n_blk)
        a01 = _sc_call(hall.reshape(K * N, D), gidx3, dst3, zeros_np,
                       n=N, d=D, n_pad=n_pad, gpt=gpt)
        h, logits = _gru_call(a01, h, W_ih, W_hh, bih2, bhh2,
                              wcls_pad, bcls_pad, n_blk=n_blk)
    return logits[:, :n_cls]


# speedup vs baseline: 2.1987x; 1.3054x over previous
import functools

import jax
import jax.numpy as jnp
from jax import lax
from jax.experimental import pallas as pl
from jax.experimental.pallas import tpu as pltpu
from jax.experimental.pallas import tpu_sc as plsc

N_STEPS = 6

NC = 2
NS = 16
CHUNK = 120


def _hall_block(h_bf, wmsg_ref, bmsg_ref, hall_ref):
    K = wmsg_ref.shape[0]
    for k in range(K):
        hall_ref[k] = lax.dot_general(
            h_bf, wmsg_ref[k], (((1,), (1,)), ((), ())),
            preferred_element_type=jnp.float32) + bmsg_ref[k]


def _hall_body(h_ref, wmsg_ref, bmsg_ref, hall_ref):
    _hall_block(h_ref[...].astype(jnp.bfloat16), wmsg_ref, bmsg_ref,
                hall_ref)


def _hall_call(h, wmsg_bf, b_msg, *, n_blk):
    N, D = h.shape
    K = wmsg_bf.shape[0]
    nb = N // n_blk
    return pl.pallas_call(
        _hall_body,
        grid=(nb,),
        in_specs=[
            pl.BlockSpec((n_blk, D), lambda i: (i, 0)),
            pl.BlockSpec((K, D, D), lambda i: (0, 0, 0)),
            pl.BlockSpec((K, 1, D), lambda i: (0, 0, 0)),
        ],
        out_specs=pl.BlockSpec((K, n_blk, D), lambda i: (0, i, 0)),
        out_shape=jax.ShapeDtypeStruct((K, N, D), jnp.float32),
    )(h, wmsg_bf, b_msg[:, None, :])


RING = 3


def _sc_body(n_pad, gpt, hall_ref, gidx_ref, dst_ref, zeros_ref,
             out_ref, idx_v, dst_v, buf_v,
             gsem0, gsem1, gsem2, ssem0, ssem1, ssem2, acc_sh):
    gsems = (gsem0, gsem1, gsem2)
    ssems = (ssem0, ssem1, ssem2)
    c = lax.axis_index("c")
    s = lax.axis_index("s")
    wid = s * NC + c

    rows_per_tile = n_pad // NS
    pltpu.sync_copy(zeros_ref.at[pl.ds(s * rows_per_tile, rows_per_tile)],
                    acc_sh.at[pl.ds(s * rows_per_tile, rows_per_tile)])
    plsc.subcore_barrier()

    def body(p, carry):
        gg = wid * gpt + p
        pr = lax.rem(p, 2)
        pltpu.sync_copy(gidx_ref.at[gg], idx_v)
        pltpu.sync_copy(dst_ref.at[gg], dst_v.at[pr])
        gathers = []
        for b in range(RING):
            @pl.when(p > 0)
            def _(b=b):
                pltpu.make_async_copy(zeros_ref.at[pl.ds(0, CHUNK)],
                                      buf_v.at[b], ssems[b]).wait()
            gathers.append(
                pltpu.async_copy(hall_ref.at[idx_v.at[b]], buf_v.at[b],
                                 gsems[b]))
        for b in range(RING):
            gathers[b].wait()
            pltpu.async_copy(buf_v.at[b], acc_sh.at[dst_v.at[pr, b]],
                             ssems[b], add=True)
        return carry

    lax.fori_loop(0, gpt, body, 0)
    for b in range(RING):
        pltpu.make_async_copy(zeros_ref.at[pl.ds(0, CHUNK)],
                              buf_v.at[b], ssems[b]).wait()
    plsc.subcore_barrier()

    pltpu.sync_copy(acc_sh.at[pl.ds(s * rows_per_tile, rows_per_tile)],
                    out_ref.at[c, pl.ds(s * rows_per_tile, rows_per_tile)])


def _sc_call(hall_flat, gidx3, dst3, zeros_np, *, n, d, n_pad, gpt):
    mesh = plsc.VectorSubcoreMesh(core_axis_name="c", subcore_axis_name="s")
    body = functools.partial(_sc_body, n_pad, gpt)
    return pl.kernel(
        body,
        out_type=jax.ShapeDtypeStruct((NC, n_pad, d), jnp.float32),
        mesh=mesh,
        scratch_types=[
            pltpu.VMEM((RING, CHUNK), jnp.int32),
            pltpu.VMEM((2, RING, CHUNK), jnp.int32),
            pltpu.VMEM((RING, CHUNK, d), jnp.float32),
            pltpu.SemaphoreType.DMA,
            pltpu.SemaphoreType.DMA,
            pltpu.SemaphoreType.DMA,
            pltpu.SemaphoreType.DMA,
            pltpu.SemaphoreType.DMA,
            pltpu.SemaphoreType.DMA,
            pltpu.VMEM_SHARED((n_pad, d), jnp.float32),
        ],
    )(hall_flat, gidx3, dst3, zeros_np)


def _gru_math(a01_ref, h_ref, wih_ref, whh_ref, bih_ref, bhh_ref):
    a = a01_ref[0] + a01_ref[1]
    h = h_ref[...]
    gi = lax.dot_general(a, wih_ref[...], (((1,), (1,)), ((), ())),
                         preferred_element_type=jnp.float32) + bih_ref[...]
    gh = lax.dot_general(h, whh_ref[...], (((1,), (1,)), ((), ())),
                         preferred_element_type=jnp.float32) + bhh_ref[...]
    D = h.shape[1]
    r = jax.nn.sigmoid(gi[:, :D] + gh[:, :D])
    z = jax.nn.sigmoid(gi[:, D:2 * D] + gh[:, D:2 * D])
    n = jnp.tanh(gi[:, 2 * D:] + r * gh[:, 2 * D:])
    return (1.0 - z) * n + z * h


def _gru_hall_body(a01_ref, h_ref, wih_ref, whh_ref, bih_ref, bhh_ref,
                   wmsg_ref, bmsg_ref, hnew_ref, hall_ref):
    hn = _gru_math(a01_ref, h_ref, wih_ref, whh_ref, bih_ref, bhh_ref)
    hnew_ref[...] = hn
    _hall_block(hn.astype(jnp.bfloat16), wmsg_ref, bmsg_ref, hall_ref)


def _gru_hall_call(a01, h, W_ih, W_hh, b_ih, b_hh, wmsg_bf, b_msg, *, n_blk):
    N, D = h.shape
    K = wmsg_bf.shape[0]
    nb = N // n_blk
    return pl.pallas_call(
        _gru_hall_body,
        grid=(nb,),
        in_specs=[
            pl.BlockSpec((2, n_blk, D), lambda i: (0, i, 0)),
            pl.BlockSpec((n_blk, D), lambda i: (i, 0)),
            pl.BlockSpec((3 * D, D), lambda i: (0, 0)),
            pl.BlockSpec((3 * D, D), lambda i: (0, 0)),
            pl.BlockSpec((1, 3 * D), lambda i: (0, 0)),
            pl.BlockSpec((1, 3 * D), lambda i: (0, 0)),
            pl.BlockSpec((K, D, D), lambda i: (0, 0, 0)),
            pl.BlockSpec((K, 1, D), lambda i: (0, 0, 0)),
        ],
        out_specs=[
            pl.BlockSpec((n_blk, D), lambda i: (i, 0)),
            pl.BlockSpec((K, n_blk, D), lambda i: (0, i, 0)),
        ],
        out_shape=[
            jax.ShapeDtypeStruct((N, D), jnp.float32),
            jax.ShapeDtypeStruct((K, N, D), jnp.float32),
        ],
    )(a01, h, W_ih, W_hh, b_ih, b_hh, wmsg_bf, b_msg[:, None, :])


def _gru_final_body(a01_ref, h_ref, wih_ref, whh_ref, bih_ref, bhh_ref,
                    wcls_ref, bcls_ref, logit_ref):
    i = pl.program_id(0)
    nb = pl.num_programs(0)
    hn = _gru_math(a01_ref, h_ref, wih_ref, whh_ref, bih_ref, bhh_ref)

    @pl.when(i == 0)
    def _():
        logit_ref[...] = jnp.zeros_like(logit_ref)

    logit_ref[...] += jnp.sum(hn, axis=0, keepdims=True)

    @pl.when(i == nb - 1)
    def _():
        hg = logit_ref[...]
        logit_ref[...] = lax.dot_general(
            hg, wcls_ref[...], (((1,), (1,)), ((), ())),
            preferred_element_type=jnp.float32) + bcls_ref[...]


def _gru_final_call(a01, h, W_ih, W_hh, b_ih, b_hh, wcls_pad, bcls_pad, *,
                    n_blk):
    N, D = h.shape
    nb = N // n_blk
    return pl.pallas_call(
        _gru_final_body,
        grid=(nb,),
        in_specs=[
            pl.BlockSpec((2, n_blk, D), lambda i: (0, i, 0)),
            pl.BlockSpec((n_blk, D), lambda i: (i, 0)),
            pl.BlockSpec((3 * D, D), lambda i: (0, 0)),
            pl.BlockSpec((3 * D, D), lambda i: (0, 0)),
            pl.BlockSpec((1, 3 * D), lambda i: (0, 0)),
            pl.BlockSpec((1, 3 * D), lambda i: (0, 0)),
            pl.BlockSpec((D, D), lambda i: (0, 0)),
            pl.BlockSpec((1, D), lambda i: (0, 0)),
        ],
        out_specs=pl.BlockSpec((1, D), lambda i: (0, 0)),
        out_shape=jax.ShapeDtypeStruct((1, D), jnp.float32),
    )(a01, h, W_ih, W_hh, b_ih, b_hh, wcls_pad, bcls_pad)


def kernel(x, edge_index, edge_type, W_msg, b_msg, W_ih, W_hh, b_ih, b_hh,
           W_cls, b_cls):
    N, D = x.shape
    K = W_msg.shape[0]
    E = edge_index.shape[1]
    n_cls = W_cls.shape[0]

    src = edge_index[0]
    dst = edge_index[1]
    gidx = edge_type * N + src

    n_workers = NC * NS
    grp = n_workers * RING * CHUNK
    e_pad = ((E + grp - 1) // grp) * grp
    gpt = e_pad // grp
    gidx = jnp.concatenate([gidx, jnp.zeros((e_pad - E,), jnp.int32)])
    dst = jnp.concatenate([dst, jnp.full((e_pad - E,), N, jnp.int32)])
    gidx3 = gidx.reshape(e_pad // (RING * CHUNK), RING, CHUNK)
    dst3 = dst.reshape(e_pad // (RING * CHUNK), RING, CHUNK)

    n_pad = ((N + 1 + NS * 8 - 1) // (NS * 8)) * (NS * 8)
    zeros_np = jnp.zeros((n_pad, D), jnp.float32)

    bih2 = b_ih.reshape(1, 3 * D)
    bhh2 = b_hh.reshape(1, 3 * D)
    wcls_pad = jnp.zeros((D, D), jnp.float32).at[:n_cls].set(W_cls)
    bcls_pad = jnp.zeros((1, D), jnp.float32).at[0, :n_cls].set(b_cls)

    n_blk = 1000
    wmsg_bf = W_msg.astype(jnp.bfloat16)
    h = x
    hall = _hall_call(h, wmsg_bf, b_msg, n_blk=n_blk)
    for _ in range(N_STEPS - 1):
        a01 = _sc_call(hall.reshape(K * N, D), gidx3, dst3, zeros_np,
                       n=N, d=D, n_pad=n_pad, gpt=gpt)
        h, hall = _gru_hall_call(a01, h, W_ih, W_hh, bih2, bhh2,
                                 wmsg_bf, b_msg, n_blk=n_blk)
    a01 = _sc_call(hall.reshape(K * N, D), gidx3, dst3, zeros_np,
                   n=N, d=D, n_pad=n_pad, gpt=gpt)
    logits = _gru_final_call(a01, h, W_ih, W_hh, bih2, bhh2,
                             wcls_pad, bcls_pad, n_blk=n_blk)
    return logits[:, :n_cls]


# speedup vs baseline: 2.3390x; 1.0638x over previous
import functools

import jax
import jax.numpy as jnp
from jax import lax
from jax.experimental import pallas as pl
from jax.experimental.pallas import tpu as pltpu
from jax.experimental.pallas import tpu_sc as plsc

N_STEPS = 6

NC = 2
NS = 16
CHUNK = 120


def _hall_block(h_bf, wmsg_ref, bmsg_ref, hall_ref):
    K = wmsg_ref.shape[0]
    for k in range(K):
        hall_ref[k] = lax.dot_general(
            h_bf, wmsg_ref[k], (((1,), (1,)), ((), ())),
            preferred_element_type=jnp.float32) + bmsg_ref[k]


def _hall_body(h_ref, wmsg_ref, bmsg_ref, hall_ref):
    _hall_block(h_ref[...].astype(jnp.bfloat16), wmsg_ref, bmsg_ref,
                hall_ref)


def _hall_call(h, wmsg_bf, b_msg, *, n_blk):
    N, D = h.shape
    K = wmsg_bf.shape[0]
    nb = N // n_blk
    return pl.pallas_call(
        _hall_body,
        grid=(nb,),
        in_specs=[
            pl.BlockSpec((n_blk, D), lambda i: (i, 0)),
            pl.BlockSpec((K, D, D), lambda i: (0, 0, 0)),
            pl.BlockSpec((K, 1, D), lambda i: (0, 0, 0)),
        ],
        out_specs=pl.BlockSpec((K, n_blk, D), lambda i: (0, i, 0)),
        out_shape=jax.ShapeDtypeStruct((K, N, D), jnp.float32),
    )(h, wmsg_bf, b_msg[:, None, :])


RING = 3


def _sc_body(n_pad, gpt, hall_ref, gidx_ref, dst_ref, zeros_ref,
             out_ref, idx_v, dst_v, buf_v,
             gsem0, gsem1, gsem2, ssem0, ssem1, ssem2, isem, acc_sh):
    gsems = (gsem0, gsem1, gsem2)
    ssems = (ssem0, ssem1, ssem2)
    c = lax.axis_index("c")
    s = lax.axis_index("s")
    wid = s * NC + c

    rows_per_tile = n_pad // NS
    pltpu.sync_copy(zeros_ref.at[pl.ds(s * rows_per_tile, rows_per_tile)],
                    acc_sh.at[pl.ds(s * rows_per_tile, rows_per_tile)])

    g0 = wid * gpt
    pltpu.async_copy(gidx_ref.at[g0], idx_v.at[0], isem)
    pltpu.async_copy(dst_ref.at[g0], dst_v.at[0], isem)
    plsc.subcore_barrier()

    def body(p, carry):
        pr2 = lax.rem(p, 2)
        pr3 = lax.rem(p, 3)
        pltpu.make_async_copy(gidx_ref.at[g0], idx_v.at[pr2], isem).wait()
        pltpu.make_async_copy(gidx_ref.at[g0], idx_v.at[pr2], isem).wait()
        @pl.when(p + 1 < gpt)
        def _():
            gg1 = wid * gpt + p + 1
            pltpu.async_copy(gidx_ref.at[gg1], idx_v.at[1 - pr2], isem)
            pltpu.async_copy(dst_ref.at[gg1],
                             dst_v.at[lax.rem(p + 1, 3)], isem)
        gathers = []
        for b in range(RING):
            @pl.when(p > 0)
            def _(b=b):
                pltpu.make_async_copy(zeros_ref.at[pl.ds(0, CHUNK)],
                                      buf_v.at[b], ssems[b]).wait()
            gathers.append(
                pltpu.async_copy(hall_ref.at[idx_v.at[pr2, b]], buf_v.at[b],
                                 gsems[b]))
        for b in range(RING):
            gathers[b].wait()
            pltpu.async_copy(buf_v.at[b], acc_sh.at[dst_v.at[pr3, b]],
                             ssems[b], add=True)
        return carry

    lax.fori_loop(0, gpt, body, 0)
    for b in range(RING):
        pltpu.make_async_copy(zeros_ref.at[pl.ds(0, CHUNK)],
                              buf_v.at[b], ssems[b]).wait()
    plsc.subcore_barrier()

    pltpu.sync_copy(acc_sh.at[pl.ds(s * rows_per_tile, rows_per_tile)],
                    out_ref.at[c, pl.ds(s * rows_per_tile, rows_per_tile)])


def _sc_call(hall_flat, gidx3, dst3, zeros_np, *, n, d, n_pad, gpt):
    mesh = plsc.VectorSubcoreMesh(core_axis_name="c", subcore_axis_name="s")
    body = functools.partial(_sc_body, n_pad, gpt)
    return pl.kernel(
        body,
        out_type=jax.ShapeDtypeStruct((NC, n_pad, d), jnp.float32),
        mesh=mesh,
        scratch_types=[
            pltpu.VMEM((2, RING, CHUNK), jnp.int32),
            pltpu.VMEM((3, RING, CHUNK), jnp.int32),
            pltpu.VMEM((RING, CHUNK, d), jnp.float32),
            pltpu.SemaphoreType.DMA,
            pltpu.SemaphoreType.DMA,
            pltpu.SemaphoreType.DMA,
            pltpu.SemaphoreType.DMA,
            pltpu.SemaphoreType.DMA,
            pltpu.SemaphoreType.DMA,
            pltpu.SemaphoreType.DMA,
            pltpu.VMEM_SHARED((n_pad, d), jnp.float32),
        ],
    )(hall_flat, gidx3, dst3, zeros_np)


def _gru_math(a01_ref, h_ref, wih_ref, whh_ref, bih_ref, bhh_ref):
    a = a01_ref[0] + a01_ref[1]
    h = h_ref[...]
    gi = lax.dot_general(a, wih_ref[...], (((1,), (1,)), ((), ())),
                         preferred_element_type=jnp.float32) + bih_ref[...]
    gh = lax.dot_general(h, whh_ref[...], (((1,), (1,)), ((), ())),
                         preferred_element_type=jnp.float32) + bhh_ref[...]
    D = h.shape[1]
    r = jax.nn.sigmoid(gi[:, :D] + gh[:, :D])
    z = jax.nn.sigmoid(gi[:, D:2 * D] + gh[:, D:2 * D])
    n = jnp.tanh(gi[:, 2 * D:] + r * gh[:, 2 * D:])
    return (1.0 - z) * n + z * h


def _gru_hall_body(a01_ref, h_ref, wih_ref, whh_ref, bih_ref, bhh_ref,
                   wmsg_ref, bmsg_ref, hnew_ref, hall_ref):
    hn = _gru_math(a01_ref, h_ref, wih_ref, whh_ref, bih_ref, bhh_ref)
    hnew_ref[...] = hn
    _hall_block(hn.astype(jnp.bfloat16), wmsg_ref, bmsg_ref, hall_ref)


def _gru_hall_call(a01, h, W_ih, W_hh, b_ih, b_hh, wmsg_bf, b_msg, *, n_blk):
    N, D = h.shape
    K = wmsg_bf.shape[0]
    nb = N // n_blk
    return pl.pallas_call(
        _gru_hall_body,
        grid=(nb,),
        in_specs=[
            pl.BlockSpec((2, n_blk, D), lambda i: (0, i, 0)),
            pl.BlockSpec((n_blk, D), lambda i: (i, 0)),
            pl.BlockSpec((3 * D, D), lambda i: (0, 0)),
            pl.BlockSpec((3 * D, D), lambda i: (0, 0)),
            pl.BlockSpec((1, 3 * D), lambda i: (0, 0)),
            pl.BlockSpec((1, 3 * D), lambda i: (0, 0)),
            pl.BlockSpec((K, D, D), lambda i: (0, 0, 0)),
            pl.BlockSpec((K, 1, D), lambda i: (0, 0, 0)),
        ],
        out_specs=[
            pl.BlockSpec((n_blk, D), lambda i: (i, 0)),
            pl.BlockSpec((K, n_blk, D), lambda i: (0, i, 0)),
        ],
        out_shape=[
            jax.ShapeDtypeStruct((N, D), jnp.float32),
            jax.ShapeDtypeStruct((K, N, D), jnp.float32),
        ],
    )(a01, h, W_ih, W_hh, b_ih, b_hh, wmsg_bf, b_msg[:, None, :])


def _gru_final_body(a01_ref, h_ref, wih_ref, whh_ref, bih_ref, bhh_ref,
                    wcls_ref, bcls_ref, logit_ref):
    i = pl.program_id(0)
    nb = pl.num_programs(0)
    hn = _gru_math(a01_ref, h_ref, wih_ref, whh_ref, bih_ref, bhh_ref)

    @pl.when(i == 0)
    def _():
        logit_ref[...] = jnp.zeros_like(logit_ref)

    logit_ref[...] += jnp.sum(hn, axis=0, keepdims=True)

    @pl.when(i == nb - 1)
    def _():
        hg = logit_ref[...]
        logit_ref[...] = lax.dot_general(
            hg, wcls_ref[...], (((1,), (1,)), ((), ())),
            preferred_element_type=jnp.float32) + bcls_ref[...]


def _gru_final_call(a01, h, W_ih, W_hh, b_ih, b_hh, wcls_pad, bcls_pad, *,
                    n_blk):
    N, D = h.shape
    nb = N // n_blk
    return pl.pallas_call(
        _gru_final_body,
        grid=(nb,),
        in_specs=[
            pl.BlockSpec((2, n_blk, D), lambda i: (0, i, 0)),
            pl.BlockSpec((n_blk, D), lambda i: (i, 0)),
            pl.BlockSpec((3 * D, D), lambda i: (0, 0)),
            pl.BlockSpec((3 * D, D), lambda i: (0, 0)),
            pl.BlockSpec((1, 3 * D), lambda i: (0, 0)),
            pl.BlockSpec((1, 3 * D), lambda i: (0, 0)),
            pl.BlockSpec((D, D), lambda i: (0, 0)),
            pl.BlockSpec((1, D), lambda i: (0, 0)),
        ],
        out_specs=pl.BlockSpec((1, D), lambda i: (0, 0)),
        out_shape=jax.ShapeDtypeStruct((1, D), jnp.float32),
    )(a01, h, W_ih, W_hh, b_ih, b_hh, wcls_pad, bcls_pad)


def kernel(x, edge_index, edge_type, W_msg, b_msg, W_ih, W_hh, b_ih, b_hh,
           W_cls, b_cls):
    N, D = x.shape
    K = W_msg.shape[0]
    E = edge_index.shape[1]
    n_cls = W_cls.shape[0]

    src = edge_index[0]
    dst = edge_index[1]
    gidx = edge_type * N + src

    n_workers = NC * NS
    grp = n_workers * RING * CHUNK
    e_pad = ((E + grp - 1) // grp) * grp
    gpt = e_pad // grp
    gidx = jnp.concatenate([gidx, jnp.zeros((e_pad - E,), jnp.int32)])
    dst = jnp.concatenate([dst, jnp.full((e_pad - E,), N, jnp.int32)])
    gidx3 = gidx.reshape(e_pad // (RING * CHUNK), RING, CHUNK)
    dst3 = dst.reshape(e_pad // (RING * CHUNK), RING, CHUNK)

    n_pad = ((N + 1 + NS * 8 - 1) // (NS * 8)) * (NS * 8)
    zeros_np = jnp.zeros((n_pad, D), jnp.float32)

    bih2 = b_ih.reshape(1, 3 * D)
    bhh2 = b_hh.reshape(1, 3 * D)
    wcls_pad = jnp.zeros((D, D), jnp.float32).at[:n_cls].set(W_cls)
    bcls_pad = jnp.zeros((1, D), jnp.float32).at[0, :n_cls].set(b_cls)

    n_blk = 1000
    wmsg_bf = W_msg.astype(jnp.bfloat16)
    h = x
    hall = _hall_call(h, wmsg_bf, b_msg, n_blk=n_blk)
    for _ in range(N_STEPS - 1):
        a01 = _sc_call(hall.reshape(K * N, D), gidx3, dst3, zeros_np,
                       n=N, d=D, n_pad=n_pad, gpt=gpt)
        h, hall = _gru_hall_call(a01, h, W_ih, W_hh, bih2, bhh2,
                                 wmsg_bf, b_msg, n_blk=n_blk)
    a01 = _sc_call(hall.reshape(K * N, D), gidx3, dst3, zeros_np,
                   n=N, d=D, n_pad=n_pad, gpt=gpt)
    logits = _gru_final_call(a01, h, W_ih, W_hh, bih2, bhh2,
                             wcls_pad, bcls_pad, n_blk=n_blk)
    return logits[:, :n_cls]
